# baseline (speedup 1.0000x reference)
import jax
import jax.numpy as jnp
from jax import lax
from jax.experimental import pallas as pl
from jax.experimental.pallas import tpu as pltpu
from jax.experimental.pallas import tpu_sc as plsc

B = 4096
H = 50
D = 64
NC = 2
NS = 16
NW = NC * NS
BW = B // NW
L = 16
NG = BW // L
NV = D // L
DP = 128

V = 100000
TC_FULL = V // DP
VP = (TC_FULL + 1) * DP
KMAX = 25
NB = 2
ROUNDS = (KMAX + NB - 1) // NB


SKEW = DP + 1


def _transpose_col(stg_b, obuf_b, lane):
  def body(d, c):
    dv = jnp.full((L,), d, jnp.int32)
    for g in range(8):
      v = stg_b[d, pl.ds(g * L, L)]
      plsc.store_scatter(obuf_b, [g * L + lane, dv], v)
    return c
  lax.fori_loop(0, D, body, 0)


def _relayout_body(ut_hbm, it_hbm, kt_hbm, utail, itail, ktail,
                   up_hbm, ip_hbm, kp_hbm,
                   stg0, stg1, ob0, ob1,
                   sem_in, sem_out):
  stg = [stg0, stg1]
  ob = [ob0, ob1]
  cid = lax.axis_index("c")
  sid = lax.axis_index("s")
  wid = sid * NC + cid
  lane = lax.iota(jnp.int32, L)

  zv = jnp.zeros((L,), jnp.float32)
  def zbody(i, c):
    for b in range(NB):
      for g in range(NV):
        ob[b][i, pl.ds(D + g * L, L)] = zv
    return c
  lax.fori_loop(0, DP, zbody, 0)

  for tbl, tail, out, tail_wid in (
      (kt_hbm, ktail, kp_hbm, 31),
      (ut_hbm, utail, up_hbm, 29),
      (it_hbm, itail, ip_hbm, 30),
  ):
    def rbody(r, c, _tbl=tbl, _out=out):
      for b in range(NB):
        k = r * NB + b
        col = wid + NW * k
        ok = jnp.logical_and(k < KMAX, col < TC_FULL)

        @pl.when(ok)
        def _fire():
          pltpu.async_copy(
              _tbl.at[:, pl.ds(col * DP, DP)], stg[b], sem_in)

      for b in range(NB):
        k = r * NB + b
        col = wid + NW * k
        ok = jnp.logical_and(k < KMAX, col < TC_FULL)
        kp_ = k - NB
        colp = wid + NW * kp_
        okp = jnp.logical_and(kp_ >= 0, colp < TC_FULL)

        @pl.when(okp)
        def _drain_prev():
          pltpu.make_async_copy(
              ob[b].at[:, pl.ds(0, DP)],
              _out.at[pl.ds(colp * DP, DP), :], sem_out).wait()

        @pl.when(ok)
        def _work():
          pltpu.make_async_copy(
              _tbl.at[:, pl.ds(col * DP, DP)], stg[b], sem_in).wait()
          _transpose_col(stg[b], ob[b], lane)
          pltpu.async_copy(
              ob[b].at[:, pl.ds(0, DP)],
              _out.at[pl.ds(col * DP, DP), :], sem_out)
      return c

    lax.fori_loop(0, ROUNDS, rbody, 0)

    for b in range(NB):
      k = (ROUNDS - 1) * NB + b
      col = wid + NW * k
      ok = jnp.logical_and(k < KMAX, col < TC_FULL)

      @pl.when(ok)
      def _drain_last(_out=out, _col=col, _b=b):
        pltpu.make_async_copy(
            ob[_b].at[:, pl.ds(0, DP)],
            _out.at[pl.ds(_col * DP, DP), :], sem_out).wait()

    @pl.when(wid == tail_wid)
    def _tail(_tail=tail, _out=out):
      pltpu.sync_copy(_tail, stg[0])
      _transpose_col(stg[0], ob[0], lane)
      pltpu.sync_copy(ob[0].at[:, pl.ds(0, DP)],
                      _out.at[pl.ds(TC_FULL * DP, DP), :])


_relayout = pl.kernel(
    _relayout_body,
    out_type=(
        jax.ShapeDtypeStruct((VP, DP), jnp.float32),
        jax.ShapeDtypeStruct((VP, DP), jnp.float32),
        jax.ShapeDtypeStruct((VP, DP), jnp.float32),
    ),
    mesh=plsc.VectorSubcoreMesh(core_axis_name="c", subcore_axis_name="s"),
    scratch_types=(
        [pltpu.VMEM((D, DP), jnp.float32)] * NB +
        [pltpu.VMEM((DP, SKEW), jnp.float32)] * NB +
        [pltpu.SemaphoreType.DMA, pltpu.SemaphoreType.DMA]),
    compiler_params=pltpu.CompilerParams(
        needs_layout_passes=False, use_tc_tiling_on_sc=True),
)


def _gather_body(user_hbm, item_hbm, kwt_hbm, up_hbm, ip_hbm, kp_hbm, out_hbm,
                 uidx, iidx, kidxt,
                 urows, irows, acc, kw0, nzf, rcpf, outv, sem):
  cid = lax.axis_index("c")
  sid = lax.axis_index("s")
  wid = sid * NC + cid
  base = wid * BW

  pltpu.sync_copy(user_hbm.at[pl.ds(base, BW)], uidx)
  pltpu.sync_copy(item_hbm.at[pl.ds(base, BW)], iidx)
  pltpu.sync_copy(kwt_hbm.at[:, pl.ds(base, BW)], kidxt)

  cp_u = pltpu.async_copy(up_hbm.at[uidx], urows, sem)
  cp_i = pltpu.async_copy(ip_hbm.at[iidx], irows, sem)
  cp_k0 = pltpu.async_copy(kp_hbm.at[kidxt.at[0]], acc, sem)
  pltpu.sync_copy(kp_hbm.at[0], kw0)

  for g in range(NG):
    def cnt_body(j, a, _g=g):
      ids = kidxt[j, pl.ds(_g * L, L)]
      return a + jnp.where(ids == 0, 1.0, 0.0)
    nz = lax.fori_loop(0, H, cnt_body, jnp.zeros((L,), jnp.float32))
    nzf[pl.ds(g * L, L)] = nz
    rcpf[pl.ds(g * L, L)] = 1.0 / jnp.maximum(jnp.float32(H) - nz, 1.0)

  cp_u.wait()
  cp_i.wait()
  cp_k0.wait()

  def fire(j, c):
    pltpu.async_copy(kp_hbm.at[kidxt.at[j]], acc, sem, add=True)
    return c
  lax.fori_loop(1, H, fire, 0)

  def drain(j, c):
    pltpu.make_async_copy(kp_hbm.at[kidxt.at[j]], acc, sem).wait()
    return c
  lax.fori_loop(1, H, drain, 0)

  lane = lax.iota(jnp.int32, L)
  last = lane == (L - 1)

  def fin(e, c):
    ev = jnp.full((L,), e, jnp.int32)
    nzv = plsc.load_gather(nzf, [ev])
    rcpv = plsc.load_gather(rcpf, [ev])
    s = jnp.zeros((L,), jnp.float32)
    for v in range(NV):
      sl = pl.ds(v * L, L)
      ic = (acc[e, sl] - nzv * kw0[sl]) * rcpv
      s = s + urows[e, sl] * (irows[e, sl] + ic)
    cs = plsc.cumsum(s)
    plsc.store_scatter(outv, [ev], cs, mask=last)
    return c

  lax.fori_loop(0, BW, fin, 0)

  pltpu.sync_copy(outv, out_hbm.at[pl.ds(base, BW)])


_gather = pl.kernel(
    _gather_body,
    out_type=jax.ShapeDtypeStruct((B,), jnp.float32),
    mesh=plsc.VectorSubcoreMesh(core_axis_name="c", subcore_axis_name="s"),
    scratch_types=[
        pltpu.VMEM((BW,), jnp.int32),
        pltpu.VMEM((BW,), jnp.int32),
        pltpu.VMEM((H, BW), jnp.int32),
        pltpu.VMEM((BW, DP), jnp.float32),
        pltpu.VMEM((BW, DP), jnp.float32),
        pltpu.VMEM((BW, DP), jnp.float32),
        pltpu.VMEM((DP,), jnp.float32),
        pltpu.VMEM((BW,), jnp.float32),
        pltpu.VMEM((BW,), jnp.float32),
        pltpu.VMEM((BW,), jnp.float32),
        pltpu.SemaphoreType.DMA,
    ],
    compiler_params=pltpu.CompilerParams(
        needs_layout_passes=False, use_tc_tiling_on_sc=False),
)


def _tail_block(t):
  return jnp.pad(t[TC_FULL * DP:], ((0, VP - V), (0, 0))).T


@jax.jit
def kernel(user, item, keyword_ids, user_emb, item_id_emb, keyword_emb):
  kw_t = keyword_ids.astype(jnp.int32).T
  up, ip, kp = _relayout(
      user_emb.T, item_id_emb.T, keyword_emb.T,
      _tail_block(user_emb), _tail_block(item_id_emb),
      _tail_block(keyword_emb))
  return _gather(user.astype(jnp.int32), item.astype(jnp.int32), kw_t,
                 up, ip, kp)


# speedup vs baseline: 1.0247x; 1.0247x over previous
import jax
import jax.numpy as jnp
from jax import lax
from jax.experimental import pallas as pl
from jax.experimental.pallas import tpu as pltpu
from jax.experimental.pallas import tpu_sc as plsc

B = 4096
H = 50
D = 64
NC = 2
NS = 16
NW = NC * NS
BW = B // NW
L = 16
NG = BW // L
NV = D // L
DP = 128

V = 100000
TC_FULL = V // DP
VP = (TC_FULL + 1) * DP
KMAX = 25
NB = 2
ROUNDS = (KMAX + NB - 1) // NB


SKEW = DP + 1


def _transpose_col(stg_b, obuf_b, lane):
  def load_row(d):
    dv = jnp.full((L,), d, jnp.int32)
    return dv, tuple(stg_b[d, pl.ds(g * L, L)] for g in range(8))

  def body(d, carry):
    dv, vs = carry
    nxt = load_row(jnp.minimum(d + 1, D - 1))
    for g in range(8):
      plsc.store_scatter(obuf_b, [g * L + lane, dv], vs[g])
    return nxt

  lax.fori_loop(0, D, body, load_row(0), unroll=2)


def _relayout_body(ut_hbm, it_hbm, kt_hbm, utail, itail, ktail,
                   up_hbm, ip_hbm, kp_hbm,
                   stg0, stg1, ob0, ob1,
                   sem_in, sem_out):
  stg = [stg0, stg1]
  ob = [ob0, ob1]
  cid = lax.axis_index("c")
  sid = lax.axis_index("s")
  wid = sid * NC + cid
  lane = lax.iota(jnp.int32, L)

  zv = jnp.zeros((L,), jnp.float32)
  def zbody(i, c):
    for b in range(NB):
      for g in range(NV):
        ob[b][i, pl.ds(D + g * L, L)] = zv
    return c
  lax.fori_loop(0, DP, zbody, 0)

  for tbl, tail, out, tail_wid in (
      (kt_hbm, ktail, kp_hbm, 31),
      (ut_hbm, utail, up_hbm, 29),
      (it_hbm, itail, ip_hbm, 30),
  ):
    def rbody(r, c, _tbl=tbl, _out=out):
      for b in range(NB):
        k = r * NB + b
        col = wid + NW * k
        ok = jnp.logical_and(k < KMAX, col < TC_FULL)

        @pl.when(ok)
        def _fire():
          pltpu.async_copy(
              _tbl.at[:, pl.ds(col * DP, DP)], stg[b], sem_in)

      for b in range(NB):
        k = r * NB + b
        col = wid + NW * k
        ok = jnp.logical_and(k < KMAX, col < TC_FULL)
        kp_ = k - NB
        colp = wid + NW * kp_
        okp = jnp.logical_and(kp_ >= 0, colp < TC_FULL)

        @pl.when(okp)
        def _drain_prev():
          pltpu.make_async_copy(
              ob[b].at[:, pl.ds(0, DP)],
              _out.at[pl.ds(colp * DP, DP), :], sem_out).wait()

        @pl.when(ok)
        def _work():
          pltpu.make_async_copy(
              _tbl.at[:, pl.ds(col * DP, DP)], stg[b], sem_in).wait()
          _transpose_col(stg[b], ob[b], lane)
          pltpu.async_copy(
              ob[b].at[:, pl.ds(0, DP)],
              _out.at[pl.ds(col * DP, DP), :], sem_out)
      return c

    lax.fori_loop(0, ROUNDS, rbody, 0)

    for b in range(NB):
      k = (ROUNDS - 1) * NB + b
      col = wid + NW * k
      ok = jnp.logical_and(k < KMAX, col < TC_FULL)

      @pl.when(ok)
      def _drain_last(_out=out, _col=col, _b=b):
        pltpu.make_async_copy(
            ob[_b].at[:, pl.ds(0, DP)],
            _out.at[pl.ds(_col * DP, DP), :], sem_out).wait()

    @pl.when(wid == tail_wid)
    def _tail(_tail=tail, _out=out):
      pltpu.sync_copy(_tail, stg[0])
      _transpose_col(stg[0], ob[0], lane)
      pltpu.sync_copy(ob[0].at[:, pl.ds(0, DP)],
                      _out.at[pl.ds(TC_FULL * DP, DP), :])


_relayout = pl.kernel(
    _relayout_body,
    out_type=(
        jax.ShapeDtypeStruct((VP, DP), jnp.float32),
        jax.ShapeDtypeStruct((VP, DP), jnp.float32),
        jax.ShapeDtypeStruct((VP, DP), jnp.float32),
    ),
    mesh=plsc.VectorSubcoreMesh(core_axis_name="c", subcore_axis_name="s"),
    scratch_types=(
        [pltpu.VMEM((D, DP), jnp.float32)] * NB +
        [pltpu.VMEM((DP, SKEW), jnp.float32)] * NB +
        [pltpu.SemaphoreType.DMA, pltpu.SemaphoreType.DMA]),
    compiler_params=pltpu.CompilerParams(
        needs_layout_passes=False, use_tc_tiling_on_sc=True),
)


def _gather_body(user_hbm, item_hbm, kwt_hbm, up_hbm, ip_hbm, kp_hbm, out_hbm,
                 uidx, iidx, kidxt,
                 urows, irows, acc, kw0, nzf, rcpf, outv, sem):
  cid = lax.axis_index("c")
  sid = lax.axis_index("s")
  wid = sid * NC + cid
  base = wid * BW

  pltpu.sync_copy(user_hbm.at[pl.ds(base, BW)], uidx)
  pltpu.sync_copy(item_hbm.at[pl.ds(base, BW)], iidx)
  pltpu.sync_copy(kwt_hbm.at[:, pl.ds(base, BW)], kidxt)

  cp_u = pltpu.async_copy(up_hbm.at[uidx], urows, sem)
  cp_i = pltpu.async_copy(ip_hbm.at[iidx], irows, sem)
  cp_k0 = pltpu.async_copy(kp_hbm.at[kidxt.at[0]], acc, sem)
  pltpu.sync_copy(kp_hbm.at[0], kw0)

  for g in range(NG):
    def cnt_body(j, a, _g=g):
      ids = kidxt[j, pl.ds(_g * L, L)]
      return a + jnp.where(ids == 0, 1.0, 0.0)
    nz = lax.fori_loop(0, H, cnt_body, jnp.zeros((L,), jnp.float32))
    nzf[pl.ds(g * L, L)] = nz
    rcpf[pl.ds(g * L, L)] = 1.0 / jnp.maximum(jnp.float32(H) - nz, 1.0)

  cp_u.wait()
  cp_i.wait()
  cp_k0.wait()

  def fire(j, c):
    pltpu.async_copy(kp_hbm.at[kidxt.at[j]], acc, sem, add=True)
    return c
  lax.fori_loop(1, H, fire, 0)

  def drain(j, c):
    pltpu.make_async_copy(kp_hbm.at[kidxt.at[j]], acc, sem).wait()
    return c
  lax.fori_loop(1, H, drain, 0)

  lane = lax.iota(jnp.int32, L)
  last = lane == (L - 1)

  def fin(e, c):
    ev = jnp.full((L,), e, jnp.int32)
    nzv = plsc.load_gather(nzf, [ev])
    rcpv = plsc.load_gather(rcpf, [ev])
    s = jnp.zeros((L,), jnp.float32)
    for v in range(NV):
      sl = pl.ds(v * L, L)
      ic = (acc[e, sl] - nzv * kw0[sl]) * rcpv
      s = s + urows[e, sl] * (irows[e, sl] + ic)
    cs = plsc.cumsum(s)
    plsc.store_scatter(outv, [ev], cs, mask=last)
    return c

  lax.fori_loop(0, BW, fin, 0)

  pltpu.sync_copy(outv, out_hbm.at[pl.ds(base, BW)])


_gather = pl.kernel(
    _gather_body,
    out_type=jax.ShapeDtypeStruct((B,), jnp.float32),
    mesh=plsc.VectorSubcoreMesh(core_axis_name="c", subcore_axis_name="s"),
    scratch_types=[
        pltpu.VMEM((BW,), jnp.int32),
        pltpu.VMEM((BW,), jnp.int32),
        pltpu.VMEM((H, BW), jnp.int32),
        pltpu.VMEM((BW, DP), jnp.float32),
        pltpu.VMEM((BW, DP), jnp.float32),
        pltpu.VMEM((BW, DP), jnp.float32),
        pltpu.VMEM((DP,), jnp.float32),
        pltpu.VMEM((BW,), jnp.float32),
        pltpu.VMEM((BW,), jnp.float32),
        pltpu.VMEM((BW,), jnp.float32),
        pltpu.SemaphoreType.DMA,
    ],
    compiler_params=pltpu.CompilerParams(
        needs_layout_passes=False, use_tc_tiling_on_sc=False),
)


def _tail_block(t):
  return jnp.pad(t[TC_FULL * DP:], ((0, VP - V), (0, 0))).T


@jax.jit
def kernel(user, item, keyword_ids, user_emb, item_id_emb, keyword_emb):
  kw_t = keyword_ids.astype(jnp.int32).T
  up, ip, kp = _relayout(
      user_emb.T, item_id_emb.T, keyword_emb.T,
      _tail_block(user_emb), _tail_block(item_id_emb),
      _tail_block(keyword_emb))
  return _gather(user.astype(jnp.int32), item.astype(jnp.int32), kw_t,
                 up, ip, kp)


# speedup vs baseline: 2.0307x; 1.9817x over previous
import jax
import jax.numpy as jnp
from jax import lax
from jax.experimental import pallas as pl
from jax.experimental.pallas import tpu as pltpu
from jax.experimental.pallas import tpu_sc as plsc

B = 4096
H = 50
D = 64
NC = 2
NS = 16
NW = NC * NS
BW = B // NW
L = 16
NG = BW // L
NV = D // L
DP = 128

V = 100000
TC_FULL = V // DP
VP = (TC_FULL + 1) * DP
KMAX = 25
NB = 2
ROUNDS = (KMAX + NB - 1) // NB


def _transpose_col(stg_b, obuf_b, lane, rots):
  for v in range(NV):
    dv = v * L + lane

    def gbody(g, c, _dv=dv):
      rb = g * L
      for k in range(L):
        rv = rb + rots[k]
        x = plsc.load_gather(stg_b, [_dv, rv])
        plsc.store_scatter(obuf_b, [rv, _dv], x)
      return c

    lax.fori_loop(0, NG, gbody, 0)


def _relayout_body(ut_hbm, it_hbm, kt_hbm, utail, itail, ktail,
                   up_hbm, ip_hbm, kp_hbm,
                   stg0, stg1, ob0, ob1,
                   sem_in, sem_out):
  stg = [stg0, stg1]
  ob = [ob0, ob1]
  cid = lax.axis_index("c")
  sid = lax.axis_index("s")
  wid = sid * NC + cid
  lane = lax.iota(jnp.int32, L)

  rots = [jnp.bitwise_and(lane + k, L - 1) for k in range(L)]

  zv = jnp.zeros((L,), jnp.float32)
  def zbody(i, c):
    for b in range(NB):
      for g in range(NV):
        ob[b][i, pl.ds(D + g * L, L)] = zv
    return c
  lax.fori_loop(0, DP, zbody, 0)

  for tbl, tail, out, tail_wid in (
      (kt_hbm, ktail, kp_hbm, 31),
      (ut_hbm, utail, up_hbm, 29),
      (it_hbm, itail, ip_hbm, 30),
  ):
    def valid(k):
      return jnp.logical_and(k < KMAX, wid + NW * k < TC_FULL)

    for b in range(NB):
      @pl.when(valid(b))
      def _prime(_tbl=tbl, _b=b):
        pltpu.async_copy(
            _tbl.at[:, pl.ds((wid + NW * _b) * DP, DP)], stg[_b], sem_in)

    def rbody(r, c, _tbl=tbl, _out=out):
      for b in range(NB):
        k = r * NB + b
        col = wid + NW * k
        kn = k + NB
        coln = wid + NW * kn
        kp_ = k - NB
        colp = wid + NW * kp_

        @pl.when(jnp.logical_and(kp_ >= 0, valid(kp_)))
        def _drain_prev():
          pltpu.make_async_copy(
              ob[b], _out.at[pl.ds(colp * DP, DP), :], sem_out).wait()

        @pl.when(valid(k))
        def _work():
          pltpu.make_async_copy(
              _tbl.at[:, pl.ds(col * DP, DP)], stg[b], sem_in).wait()
          _transpose_col(stg[b], ob[b], lane, rots)

          @pl.when(valid(kn))
          def _prefetch():
            pltpu.async_copy(
                _tbl.at[:, pl.ds(coln * DP, DP)], stg[b], sem_in)

          pltpu.async_copy(
              ob[b], _out.at[pl.ds(col * DP, DP), :], sem_out)
      return c

    lax.fori_loop(0, ROUNDS, rbody, 0)

    for b in range(NB):
      k = (ROUNDS - 1) * NB + b
      col = wid + NW * k

      @pl.when(valid(k))
      def _drain_last(_out=out, _col=col, _b=b):
        pltpu.make_async_copy(
            ob[_b], _out.at[pl.ds(_col * DP, DP), :], sem_out).wait()

    @pl.when(wid == tail_wid)
    def _tail(_tail=tail, _out=out):
      pltpu.sync_copy(_tail, stg[0])
      _transpose_col(stg[0], ob[0], lane, rots)
      pltpu.sync_copy(ob[0], _out.at[pl.ds(TC_FULL * DP, DP), :])


_relayout = pl.kernel(
    _relayout_body,
    out_type=(
        jax.ShapeDtypeStruct((VP, DP), jnp.float32),
        jax.ShapeDtypeStruct((VP, DP), jnp.float32),
        jax.ShapeDtypeStruct((VP, DP), jnp.float32),
    ),
    mesh=plsc.VectorSubcoreMesh(core_axis_name="c", subcore_axis_name="s"),
    scratch_types=(
        [pltpu.VMEM((D, DP), jnp.float32)] * NB +
        [pltpu.VMEM((DP, DP), jnp.float32)] * NB +
        [pltpu.SemaphoreType.DMA, pltpu.SemaphoreType.DMA]),
    compiler_params=pltpu.CompilerParams(
        needs_layout_passes=False, use_tc_tiling_on_sc=True),
)


def _gather_body(user_hbm, item_hbm, kwt_hbm, up_hbm, ip_hbm, kp_hbm, out_hbm,
                 uidx, iidx, kidxt,
                 urows, irows, acc, kw0, nzf, rcpf, outv, sem):
  cid = lax.axis_index("c")
  sid = lax.axis_index("s")
  wid = sid * NC + cid
  base = wid * BW

  pltpu.sync_copy(user_hbm.at[pl.ds(base, BW)], uidx)
  pltpu.sync_copy(item_hbm.at[pl.ds(base, BW)], iidx)
  pltpu.sync_copy(kwt_hbm.at[:, pl.ds(base, BW)], kidxt)

  cp_u = pltpu.async_copy(up_hbm.at[uidx], urows, sem)
  cp_i = pltpu.async_copy(ip_hbm.at[iidx], irows, sem)
  cp_k0 = pltpu.async_copy(kp_hbm.at[kidxt.at[0]], acc, sem)
  pltpu.sync_copy(kp_hbm.at[0], kw0)

  for g in range(NG):
    def cnt_body(j, a, _g=g):
      ids = kidxt[j, pl.ds(_g * L, L)]
      return a + jnp.where(ids == 0, 1.0, 0.0)
    nz = lax.fori_loop(0, H, cnt_body, jnp.zeros((L,), jnp.float32))
    nzf[pl.ds(g * L, L)] = nz
    rcpf[pl.ds(g * L, L)] = 1.0 / jnp.maximum(jnp.float32(H) - nz, 1.0)

  cp_u.wait()
  cp_i.wait()
  cp_k0.wait()

  def fire(j, c):
    pltpu.async_copy(kp_hbm.at[kidxt.at[j]], acc, sem, add=True)
    return c
  lax.fori_loop(1, H, fire, 0)

  def drain(j, c):
    pltpu.make_async_copy(kp_hbm.at[kidxt.at[j]], acc, sem).wait()
    return c
  lax.fori_loop(1, H, drain, 0)

  lane = lax.iota(jnp.int32, L)
  last = lane == (L - 1)

  def fin(e, c):
    ev = jnp.full((L,), e, jnp.int32)
    nzv = plsc.load_gather(nzf, [ev])
    rcpv = plsc.load_gather(rcpf, [ev])
    s = jnp.zeros((L,), jnp.float32)
    for v in range(NV):
      sl = pl.ds(v * L, L)
      ic = (acc[e, sl] - nzv * kw0[sl]) * rcpv
      s = s + urows[e, sl] * (irows[e, sl] + ic)
    cs = plsc.cumsum(s)
    plsc.store_scatter(outv, [ev], cs, mask=last)
    return c

  lax.fori_loop(0, BW, fin, 0)

  pltpu.sync_copy(outv, out_hbm.at[pl.ds(base, BW)])


_gather = pl.kernel(
    _gather_body,
    out_type=jax.ShapeDtypeStruct((B,), jnp.float32),
    mesh=plsc.VectorSubcoreMesh(core_axis_name="c", subcore_axis_name="s"),
    scratch_types=[
        pltpu.VMEM((BW,), jnp.int32),
        pltpu.VMEM((BW,), jnp.int32),
        pltpu.VMEM((H, BW), jnp.int32),
        pltpu.VMEM((BW, DP), jnp.float32),
        pltpu.VMEM((BW, DP), jnp.float32),
        pltpu.VMEM((BW, DP), jnp.float32),
        pltpu.VMEM((DP,), jnp.float32),
        pltpu.VMEM((BW,), jnp.float32),
        pltpu.VMEM((BW,), jnp.float32),
        pltpu.VMEM((BW,), jnp.float32),
        pltpu.SemaphoreType.DMA,
    ],
    compiler_params=pltpu.CompilerParams(
        needs_layout_passes=False, use_tc_tiling_on_sc=False),
)


def _tail_block(t):
  return jnp.pad(t[TC_FULL * DP:], ((0, VP - V), (0, 0))).T


@jax.jit
def kernel(user, item, keyword_ids, user_emb, item_id_emb, keyword_emb):
  kw_t = keyword_ids.astype(jnp.int32).T
  up, ip, kp = _relayout(
      user_emb.T, item_id_emb.T, keyword_emb.T,
      _tail_block(user_emb), _tail_block(item_id_emb),
      _tail_block(keyword_emb))
  return _gather(user.astype(jnp.int32), item.astype(jnp.int32), kw_t,
                 up, ip, kp)
